# Initial kernel scaffold; baseline (speedup 1.0000x reference)
import jax
import jax.numpy as jnp
from jax.experimental import pallas as pl


def kernel(qx, kx, vx, WQ_w, WQ_b, WK_w, WK_b, WV_w, WV_b, WO_w, WO_b):
    raise NotImplementedError("write your pallas kernel here")


# speedup vs baseline: 89.8939x; 89.8939x over previous
import jax
import jax.numpy as jnp
from jax.experimental import pallas as pl

FEA = 768
DK = 64
H = 12
L = 2048
SCALE = 1.0 / 8.0
BL = 256
NB = L // BL


def _head_onehot():
    ci = jax.lax.broadcasted_iota(jnp.int32, (FEA, H), 0)
    hi = jax.lax.broadcasted_iota(jnp.int32, (FEA, H), 1)
    return (ci // DK == hi).astype(jnp.float32)


def _head_onehot_t():
    hi = jax.lax.broadcasted_iota(jnp.int32, (H, FEA), 0)
    ci = jax.lax.broadcasted_iota(jnp.int32, (H, FEA), 1)
    return (ci // DK == hi).astype(jnp.float32)


def _mm_t(x, w):
    return jax.lax.dot_general(x, w, (((1,), (1,)), ((), ())),
                               preferred_element_type=jnp.float32)


def _mm(x, w):
    return jax.lax.dot_general(x, w, (((1,), (0,)), ((), ())),
                               preferred_element_type=jnp.float32)


def _proj_body(qx, kx, vx, wq, wk, wv, bq, bk, bv, qo, ko, vo):
    qo[...] = _mm_t(qx[...], wq[...]) + bq[...]
    ko[...] = _mm_t(kx[...], wk[...]) + bk[...]
    vo[...] = _mm_t(vx[...], wv[...]) + bv[...]


def _attn_body(qref, kref, vref, wo, bo, out):
    j = pl.program_id(0)
    base = j * BL
    E = _head_onehot()
    ET = _head_onehot_t()

    Qb = qref[...] * SCALE
    Kf = kref[...]
    Vf = vref[...]

    k0 = Kf[0:1, :]
    kL = Kf[L - 1:L, :]
    v0 = Vf[0:1, :]
    vL = Vf[L - 1:L, :]
    sall = jnp.sum(Vf, axis=0, keepdims=True)

    kblk = kref[pl.ds(base, BL), :]
    vblk = vref[pl.ds(base, BL), :]
    kprev = kref[pl.ds(jnp.maximum(base - 1, 0), 1), :]
    knext = kref[pl.ds(jnp.minimum(base + BL, L - 1), 1), :]
    vprev = vref[pl.ds(jnp.maximum(base - 1, 0), 1), :]
    vnext = vref[pl.ds(jnp.minimum(base + BL, L - 1), 1), :]
    km1 = jnp.concatenate([kprev, kblk[:BL - 1, :]], axis=0)
    kp1 = jnp.concatenate([kblk[1:, :], knext], axis=0)
    vm1 = jnp.concatenate([vprev, vblk[:BL - 1, :]], axis=0)
    vp1 = jnp.concatenate([vblk[1:, :], vnext], axis=0)

    e0 = _mm(Qb * k0, E)
    eL = _mm(Qb * kL, E)
    ed = _mm(Qb * kblk, E)
    esub = _mm(Qb * km1, E)
    esup = _mm(Qb * kp1, E)

    x0 = jnp.exp(e0)
    xL = jnp.exp(eL)
    xd = jnp.exp(ed)
    xsub = jnp.exp(esub)
    xsup = jnp.exp(esup)

    gi = base + jax.lax.broadcasted_iota(jnp.int32, (BL, 1), 0)
    msub = (gi != 1).astype(jnp.float32)
    msup = (gi != L - 2).astype(jnp.float32)

    denom = (x0 + xL + xd + msub * xsub + msup * xsup
             + (jnp.float32(L - 3) - msub - msup))

    num = (sall
           + _mm(x0 - 1.0, ET) * v0
           + _mm(xL - 1.0, ET) * vL
           + _mm(xd - 1.0, ET) * vblk
           + _mm(msub * (xsub - 1.0), ET) * vm1
           + _mm(msup * (xsup - 1.0), ET) * vp1)
    z = num / _mm(denom, ET)

    out[...] = _mm_t(z, wo[...]) + bo[...]

    @pl.when(j == 0)
    def _():
        s0 = _mm(Kf * Qb[0:1, :], E)
        a0 = jnp.exp(s0 - jnp.max(s0, axis=0, keepdims=True))
        alpha0 = a0 / jnp.sum(a0, axis=0, keepdims=True)
        z0 = jnp.sum(_mm(alpha0, ET) * Vf, axis=0, keepdims=True)
        out[0:1, :] = _mm_t(z0, wo[...]) + bo[...]

    @pl.when(j == NB - 1)
    def _():
        sL = _mm(Kf * Qb[BL - 1:BL, :], E)
        aL = jnp.exp(sL - jnp.max(sL, axis=0, keepdims=True))
        alphaL = aL / jnp.sum(aL, axis=0, keepdims=True)
        zL = jnp.sum(_mm(alphaL, ET) * Vf, axis=0, keepdims=True)
        out[BL - 1:BL, :] = _mm_t(zL, wo[...]) + bo[...]


def kernel(qx, kx, vx, WQ_w, WQ_b, WK_w, WK_b, WV_w, WV_b, WO_w, WO_b):
    q2 = qx.reshape(L, FEA)
    k2 = kx.reshape(L, FEA)
    v2 = vx.reshape(L, FEA)
    bq = WQ_b.reshape(1, FEA)
    bk = WK_b.reshape(1, FEA)
    bv = WV_b.reshape(1, FEA)
    bo = WO_b.reshape(1, FEA)

    blk = pl.BlockSpec((BL, FEA), lambda j: (j, 0))
    full_w = pl.BlockSpec((FEA, FEA), lambda j: (0, 0))
    full_b = pl.BlockSpec((1, FEA), lambda j: (0, 0))
    full_seq = pl.BlockSpec((L, FEA), lambda j: (0, 0))

    Q, K, V = pl.pallas_call(
        _proj_body,
        grid=(NB,),
        in_specs=[blk, blk, blk, full_w, full_w, full_w,
                  full_b, full_b, full_b],
        out_specs=[blk, blk, blk],
        out_shape=[jax.ShapeDtypeStruct((L, FEA), jnp.float32)] * 3,
    )(q2, k2, v2, WQ_w, WK_w, WV_w, bq, bk, bv)

    out = pl.pallas_call(
        _attn_body,
        grid=(NB,),
        in_specs=[blk, full_seq, full_seq, full_w, full_b],
        out_specs=blk,
        out_shape=jax.ShapeDtypeStruct((L, FEA), jnp.float32),
    )(Q, K, V, WO_w, bo)

    return out.reshape(1, L, FEA)


# speedup vs baseline: 114.9537x; 1.2788x over previous
import jax
import jax.numpy as jnp
from jax.experimental import pallas as pl
from jax.experimental.pallas import tpu as pltpu

FEA = 768
DK = 64
H = 12
L = 2048
SCALE = 1.0 / 8.0
BL = 256
NB = L // BL


def _head_onehot():
    ci = jax.lax.broadcasted_iota(jnp.int32, (FEA, H), 0)
    hi = jax.lax.broadcasted_iota(jnp.int32, (FEA, H), 1)
    return (ci // DK == hi).astype(jnp.float32)


def _head_onehot_t():
    hi = jax.lax.broadcasted_iota(jnp.int32, (H, FEA), 0)
    ci = jax.lax.broadcasted_iota(jnp.int32, (H, FEA), 1)
    return (ci // DK == hi).astype(jnp.float32)


def _mm_t(x, w):
    return jax.lax.dot_general(x, w, (((1,), (1,)), ((), ())),
                               preferred_element_type=jnp.float32)


def _mm(x, w):
    return jax.lax.dot_general(x, w, (((1,), (0,)), ((), ())),
                               preferred_element_type=jnp.float32)


def _body(qx, kx, vx, wq, wk, wv, wo, bq, bk, bv, bo, out, Qs, Ks, Vs, sall_s):
    p = pl.program_id(0)
    j = pl.program_id(1)
    base = j * BL

    @pl.when(p == 0)
    def _proj():
        Qs[pl.ds(base, BL), :] = (_mm_t(qx[...], wq[...]) + bq[...]) * SCALE
        Ks[pl.ds(base, BL), :] = _mm_t(kx[...], wk[...]) + bk[...]
        vv = _mm_t(vx[...], wv[...]) + bv[...]
        Vs[pl.ds(base, BL), :] = vv
        part = jnp.sum(vv, axis=0, keepdims=True)

        @pl.when(j == 0)
        def _():
            sall_s[...] = part

        @pl.when(j > 0)
        def _():
            sall_s[...] += part

    @pl.when(p == 1)
    def _attn():
        E = _head_onehot()
        ET = _head_onehot_t()

        Qb = Qs[pl.ds(base, BL), :]
        k0 = Ks[0:1, :]
        kL = Ks[L - 1:L, :]
        v0 = Vs[0:1, :]
        vL = Vs[L - 1:L, :]
        sall = sall_s[...]

        kblk = Ks[pl.ds(base, BL), :]
        vblk = Vs[pl.ds(base, BL), :]
        kprev = Ks[pl.ds(jnp.maximum(base - 1, 0), 1), :]
        knext = Ks[pl.ds(jnp.minimum(base + BL, L - 1), 1), :]
        vprev = Vs[pl.ds(jnp.maximum(base - 1, 0), 1), :]
        vnext = Vs[pl.ds(jnp.minimum(base + BL, L - 1), 1), :]
        km1 = jnp.concatenate([kprev, kblk[:BL - 1, :]], axis=0)
        kp1 = jnp.concatenate([kblk[1:, :], knext], axis=0)
        vm1 = jnp.concatenate([vprev, vblk[:BL - 1, :]], axis=0)
        vp1 = jnp.concatenate([vblk[1:, :], vnext], axis=0)

        x0 = jnp.exp(_mm(Qb * k0, E))
        xL = jnp.exp(_mm(Qb * kL, E))
        xd = jnp.exp(_mm(Qb * kblk, E))
        xsub = jnp.exp(_mm(Qb * km1, E))
        xsup = jnp.exp(_mm(Qb * kp1, E))

        gi = base + jax.lax.broadcasted_iota(jnp.int32, (BL, 1), 0)
        msub = (gi != 1).astype(jnp.float32)
        msup = (gi != L - 2).astype(jnp.float32)

        denom = (x0 + xL + xd + msub * xsub + msup * xsup
                 + (jnp.float32(L - 3) - msub - msup))

        num = (sall
               + _mm(x0 - 1.0, ET) * v0
               + _mm(xL - 1.0, ET) * vL
               + _mm(xd - 1.0, ET) * vblk
               + _mm(msub * (xsub - 1.0), ET) * vm1
               + _mm(msup * (xsup - 1.0), ET) * vp1)
        z = num / _mm(denom, ET)

        out[...] = _mm_t(z, wo[...]) + bo[...]

        @pl.when(j == 0)
        def _():
            s0 = _mm(Ks[...] * Qb[0:1, :], E)
            a0 = jnp.exp(s0 - jnp.max(s0, axis=0, keepdims=True))
            alpha0 = a0 / jnp.sum(a0, axis=0, keepdims=True)
            z0 = jnp.sum(_mm(alpha0, ET) * Vs[...], axis=0, keepdims=True)
            out[0:1, :] = _mm_t(z0, wo[...]) + bo[...]

        @pl.when(j == NB - 1)
        def _():
            sL = _mm(Ks[...] * Qb[BL - 1:BL, :], E)
            aL = jnp.exp(sL - jnp.max(sL, axis=0, keepdims=True))
            alphaL = aL / jnp.sum(aL, axis=0, keepdims=True)
            zL = jnp.sum(_mm(alphaL, ET) * Vs[...], axis=0, keepdims=True)
            out[BL - 1:BL, :] = _mm_t(zL, wo[...]) + bo[...]


def kernel(qx, kx, vx, WQ_w, WQ_b, WK_w, WK_b, WV_w, WV_b, WO_w, WO_b):
    q2 = qx.reshape(L, FEA)
    k2 = kx.reshape(L, FEA)
    v2 = vx.reshape(L, FEA)
    bq = WQ_b.reshape(1, FEA)
    bk = WK_b.reshape(1, FEA)
    bv = WV_b.reshape(1, FEA)
    bo = WO_b.reshape(1, FEA)

    in_blk = pl.BlockSpec((BL, FEA), lambda p, j: (j * (1 - p), 0))
    full_w = pl.BlockSpec((FEA, FEA), lambda p, j: (0, 0))
    full_b = pl.BlockSpec((1, FEA), lambda p, j: (0, 0))
    out_blk = pl.BlockSpec((BL, FEA), lambda p, j: (j * p, 0))

    out = pl.pallas_call(
        _body,
        grid=(2, NB),
        in_specs=[in_blk, in_blk, in_blk, full_w, full_w, full_w, full_w,
                  full_b, full_b, full_b, full_b],
        out_specs=out_blk,
        out_shape=jax.ShapeDtypeStruct((L, FEA), jnp.float32),
        scratch_shapes=[
            pltpu.VMEM((L, FEA), jnp.float32),
            pltpu.VMEM((L, FEA), jnp.float32),
            pltpu.VMEM((L, FEA), jnp.float32),
            pltpu.VMEM((1, FEA), jnp.float32),
        ],
    )(q2, k2, v2, WQ_w, WK_w, WV_w, WO_w, bq, bk, bv, bo)

    return out.reshape(1, L, FEA)


# speedup vs baseline: 122.6413x; 1.0669x over previous
import jax
import jax.numpy as jnp
from jax.experimental import pallas as pl
from jax.experimental.pallas import tpu as pltpu

FEA = 768
DK = 64
H = 12
L = 2048
SCALE = 1.0 / 8.0
BL = 512
NB = L // BL


def _head_onehot():
    ci = jax.lax.broadcasted_iota(jnp.int32, (FEA, H), 0)
    hi = jax.lax.broadcasted_iota(jnp.int32, (FEA, H), 1)
    return (ci // DK == hi).astype(jnp.float32)


def _head_onehot_t():
    hi = jax.lax.broadcasted_iota(jnp.int32, (H, FEA), 0)
    ci = jax.lax.broadcasted_iota(jnp.int32, (H, FEA), 1)
    return (ci // DK == hi).astype(jnp.float32)


def _mm_t(x, w):
    return jax.lax.dot_general(x, w, (((1,), (1,)), ((), ())),
                               preferred_element_type=jnp.float32)


def _mm(x, w):
    return jax.lax.dot_general(x, w, (((1,), (0,)), ((), ())),
                               preferred_element_type=jnp.float32)


def _body(qx, kx, vx, wq, wk, wv, wo, bq, bk, bv, bo, out, Qs, Ks, Vs, sall_s):
    p = pl.program_id(0)
    j = pl.program_id(1)
    base = j * BL

    @pl.when(p == 0)
    def _proj():
        Qs[pl.ds(base, BL), :] = (_mm_t(qx[...], wq[...]) + bq[...]) * SCALE
        Ks[pl.ds(base, BL), :] = _mm_t(kx[...], wk[...]) + bk[...]
        vv = _mm_t(vx[...], wv[...]) + bv[...]
        Vs[pl.ds(base, BL), :] = vv
        part = jnp.sum(vv, axis=0, keepdims=True)

        @pl.when(j == 0)
        def _():
            sall_s[...] = part

        @pl.when(j > 0)
        def _():
            sall_s[...] += part

    @pl.when(p == 1)
    def _attn():
        E = _head_onehot()
        ET = _head_onehot_t()

        Qb = Qs[pl.ds(base, BL), :]
        k0 = Ks[0:1, :]
        kL = Ks[L - 1:L, :]
        v0 = Vs[0:1, :]
        vL = Vs[L - 1:L, :]
        sall = sall_s[...]

        kblk = Ks[pl.ds(base, BL), :]
        vblk = Vs[pl.ds(base, BL), :]
        kprev = Ks[pl.ds(jnp.maximum(base - 1, 0), 1), :]
        knext = Ks[pl.ds(jnp.minimum(base + BL, L - 1), 1), :]
        vprev = Vs[pl.ds(jnp.maximum(base - 1, 0), 1), :]
        vnext = Vs[pl.ds(jnp.minimum(base + BL, L - 1), 1), :]
        km1 = jnp.concatenate([kprev, kblk[:BL - 1, :]], axis=0)
        kp1 = jnp.concatenate([kblk[1:, :], knext], axis=0)
        vm1 = jnp.concatenate([vprev, vblk[:BL - 1, :]], axis=0)
        vp1 = jnp.concatenate([vblk[1:, :], vnext], axis=0)

        x0 = jnp.exp(_mm(Qb * k0, E))
        xL = jnp.exp(_mm(Qb * kL, E))
        xd = jnp.exp(_mm(Qb * kblk, E))
        xsub = jnp.exp(_mm(Qb * km1, E))
        xsup = jnp.exp(_mm(Qb * kp1, E))

        gi = base + jax.lax.broadcasted_iota(jnp.int32, (BL, 1), 0)
        msub = (gi != 1).astype(jnp.float32)
        msup = (gi != L - 2).astype(jnp.float32)

        denom = (x0 + xL + xd + msub * xsub + msup * xsup
                 + (jnp.float32(L - 3) - msub - msup))

        num = (sall
               + _mm(x0 - 1.0, ET) * v0
               + _mm(xL - 1.0, ET) * vL
               + _mm(xd - 1.0, ET) * vblk
               + _mm(msub * (xsub - 1.0), ET) * vm1
               + _mm(msup * (xsup - 1.0), ET) * vp1)
        z = num / _mm(denom, ET)

        out[...] = _mm_t(z, wo[...]) + bo[...]

        @pl.when(j == 0)
        def _():
            s0 = _mm(Ks[...] * Qb[0:1, :], E)
            a0 = jnp.exp(s0 - jnp.max(s0, axis=0, keepdims=True))
            alpha0 = a0 / jnp.sum(a0, axis=0, keepdims=True)
            z0 = jnp.sum(_mm(alpha0, ET) * Vs[...], axis=0, keepdims=True)
            out[0:1, :] = _mm_t(z0, wo[...]) + bo[...]

        @pl.when(j == NB - 1)
        def _():
            sL = _mm(Ks[...] * Qb[BL - 1:BL, :], E)
            aL = jnp.exp(sL - jnp.max(sL, axis=0, keepdims=True))
            alphaL = aL / jnp.sum(aL, axis=0, keepdims=True)
            zL = jnp.sum(_mm(alphaL, ET) * Vs[...], axis=0, keepdims=True)
            out[BL - 1:BL, :] = _mm_t(zL, wo[...]) + bo[...]


def kernel(qx, kx, vx, WQ_w, WQ_b, WK_w, WK_b, WV_w, WV_b, WO_w, WO_b):
    q2 = qx.reshape(L, FEA)
    k2 = kx.reshape(L, FEA)
    v2 = vx.reshape(L, FEA)
    bq = WQ_b.reshape(1, FEA)
    bk = WK_b.reshape(1, FEA)
    bv = WV_b.reshape(1, FEA)
    bo = WO_b.reshape(1, FEA)

    in_blk = pl.BlockSpec((BL, FEA), lambda p, j: (j * (1 - p), 0))
    full_w = pl.BlockSpec((FEA, FEA), lambda p, j: (0, 0))
    full_b = pl.BlockSpec((1, FEA), lambda p, j: (0, 0))
    out_blk = pl.BlockSpec((BL, FEA), lambda p, j: (j * p, 0))

    out = pl.pallas_call(
        _body,
        grid=(2, NB),
        in_specs=[in_blk, in_blk, in_blk, full_w, full_w, full_w, full_w,
                  full_b, full_b, full_b, full_b],
        out_specs=out_blk,
        out_shape=jax.ShapeDtypeStruct((L, FEA), jnp.float32),
        scratch_shapes=[
            pltpu.VMEM((L, FEA), jnp.float32),
            pltpu.VMEM((L, FEA), jnp.float32),
            pltpu.VMEM((L, FEA), jnp.float32),
            pltpu.VMEM((1, FEA), jnp.float32),
        ],
    )(q2, k2, v2, WQ_w, WK_w, WV_w, WO_w, bq, bk, bv, bo)

    return out.reshape(1, L, FEA)
